# baseline (speedup 1.0000x reference)
import functools

import jax
import jax.numpy as jnp
from jax import lax
from jax.experimental import pallas as pl
from jax.experimental.pallas import tpu as pltpu
from jax.experimental.pallas import tpu_sc as plsc

N = 10000
NP = 10240
E = 160000
NC = 2
NS = 16
DH = 128

RPT = NP // NS
K = 48
EP = 161280
KD = 40
RB = 5

_MESH = dict(core_axis_name="c", subcore_axis_name="s")


def _prop_pipeline(nchunk, zrows, dummy_idx, acc, sd_idx, rows,
                   isems, gsems, ssems, idx_load, gather):
    def gwait(b):
        pltpu.make_async_copy(zrows, rows[b], gsems[b]).wait()

    def iwait(b):
        pltpu.make_async_copy(dummy_idx, sd_idx[b], isems[b]).wait()

    def swait(b):
        pltpu.make_async_copy(zrows, rows[b], ssems[b]).wait()

    for t in range(3):
        idx_load(t, t)
    for t in range(2):
        iwait(t)
        gather(t, t)

    def outer(ii, carry):
        for b in range(RB):
            j = ii * RB + b
            gwait(b)
            pltpu.async_copy(rows[b], acc.at[sd_idx[b].at[1]], ssems[b],
                             add=True)

            @pl.when(j >= 2)
            def _():
                swait((b - 2) % RB)

            @pl.when(j + 3 < nchunk)
            def _():
                idx_load(j + 3, (b + 3) % RB)

            @pl.when(j + 2 < nchunk)
            def _():
                iwait((b + 2) % RB)
                gather(j + 2, (b + 2) % RB)

            return_val = carry
        return return_val

    lax.fori_loop(0, nchunk // RB, outer, 0)
    swait((nchunk - 2) % RB)
    swait((nchunk - 1) % RB)


def _zero_acc(zrows, rows0, acc, row0, sem):
    pltpu.sync_copy(zrows, rows0)
    nz = RPT // K
    for jj in range(nz):
        pltpu.async_copy(rows0, acc.at[pl.ds(row0 + jj * K, K)], sem)
    for jj in range(nz):
        pltpu.make_async_copy(zrows, rows0, sem).wait()


def _writeout(zrows, acc, rows, row0, cid, out0, out1, wsems):
    nw = RPT // K
    for jj in range(nw):
        b = jj & 1
        sl = pl.ds(row0 + jj * K, K)
        if jj >= 2:
            pltpu.make_async_copy(zrows, rows[b], wsems[b]).wait()
        pltpu.sync_copy(acc.at[sl], rows[b])

        @pl.when(cid == 0)
        def _():
            pltpu.async_copy(rows[b], out0.at[sl], wsems[b])

        @pl.when(cid == 1)
        def _():
            pltpu.async_copy(rows[b], out1.at[sl], wsems[b])

    for b in range(2):
        pltpu.make_async_copy(zrows, rows[b], wsems[b]).wait()


def _sc_scratch(nbuf=RB):
    return [
        pltpu.VMEM_SHARED((NP, DH), jnp.float32),
        [pltpu.VMEM((2, K), jnp.int32)] * nbuf,
        [pltpu.VMEM((K, DH), jnp.float32)] * nbuf,
        [pltpu.SemaphoreType.DMA] * nbuf,
        [pltpu.SemaphoreType.DMA] * nbuf,
        [pltpu.SemaphoreType.DMA] * nbuf,
    ]


_OUT2 = (
    jax.ShapeDtypeStruct((NP, DH), jnp.float32),
    jax.ShapeDtypeStruct((NP, DH), jnp.float32),
)


@functools.lru_cache(maxsize=None)
def _make_prop_col():
    EPT = EP // NS
    NCHUNK = EPT // K

    @functools.partial(
        pl.kernel, out_type=_OUT2, mesh=plsc.VectorSubcoreMesh(**_MESH),
        scratch_types=_sc_scratch(),
    )
    def prop(xa, xb, sd4, zrows, outa, outb,
             acc, sd_idx, rows, isems, gsems, ssems):
        tid = lax.axis_index("s")
        cid = lax.axis_index("c")
        row0 = tid * RPT

        _zero_acc(zrows, rows[0], acc, row0, ssems[0])
        plsc.subcore_barrier()

        def idx_load(j, b):
            pltpu.async_copy(sd4.at[tid, j], sd_idx[b], isems[b])

        def gather(j, b):
            @pl.when(cid == 0)
            def _():
                pltpu.async_copy(xa.at[sd_idx[b].at[0]], rows[b], gsems[b])

            @pl.when(cid == 1)
            def _():
                pltpu.async_copy(xb.at[sd_idx[b].at[0]], rows[b], gsems[b])

        _prop_pipeline(NCHUNK, zrows, sd4.at[0, 0], acc, sd_idx,
                       rows, isems, gsems, ssems, idx_load, gather)
        plsc.subcore_barrier()
        _writeout(zrows, acc, rows, row0, cid, outa, outb, gsems)

    return prop


@functools.lru_cache(maxsize=None)
def _make_prop_edge():
    EPT = EP // (NC * NS)
    NCHUNK = EPT // K

    @functools.partial(
        pl.kernel, out_type=_OUT2, mesh=plsc.VectorSubcoreMesh(**_MESH),
        scratch_types=_sc_scratch(),
    )
    def prop(x, sd4, zrows, out0, out1,
             acc, sd_idx, rows, isems, gsems, ssems):
        tid = lax.axis_index("s")
        cid = lax.axis_index("c")
        row0 = tid * RPT
        wid = cid * NS + tid

        _zero_acc(zrows, rows[0], acc, row0, ssems[0])
        plsc.subcore_barrier()

        def idx_load(j, b):
            pltpu.async_copy(sd4.at[wid, j], sd_idx[b], isems[b])

        def gather(j, b):
            pltpu.async_copy(x.at[sd_idx[b].at[0]], rows[b], gsems[b])

        _prop_pipeline(NCHUNK, zrows, sd4.at[0, 0], acc, sd_idx,
                       rows, isems, gsems, ssems, idx_load, gather)
        plsc.subcore_barrier()
        _writeout(zrows, acc, rows, row0, cid, out0, out1, gsems)

    return prop


@functools.lru_cache(maxsize=None)
def _make_deg():
    EPT = E // (NC * NS)
    NCHUNK = EPT // KD

    @functools.partial(
        pl.kernel, out_type=_OUT2, mesh=plsc.VectorSubcoreMesh(**_MESH),
        scratch_types=[
            pltpu.VMEM_SHARED((NP, DH), jnp.float32),
            pltpu.VMEM((NCHUNK, KD), jnp.int32),
            pltpu.VMEM((K, DH), jnp.float32),
            pltpu.VMEM((K, DH), jnp.float32),
            pltpu.VMEM((KD, DH), jnp.float32),
            [pltpu.SemaphoreType.DMA] * 4,
        ],
    )
    def deg(src3, ones_hbm, zrows, out0, out1,
            acc, src_v, ones_v, pbuf, ones_k, sems):
        tid = lax.axis_index("s")
        cid = lax.axis_index("c")
        row0 = tid * RPT
        wid = cid * NS + tid

        _zero_acc(zrows, ones_v, acc, row0, sems[0])
        pltpu.sync_copy(src3.at[wid], src_v)
        pltpu.sync_copy(ones_hbm, ones_k)
        plsc.subcore_barrier()

        def swait(b):
            pltpu.make_async_copy(ones_hbm, ones_k, sems[b]).wait()

        def outer(ii, carry):
            for b in range(4):
                j = ii * 4 + b

                @pl.when(j >= 4)
                def _():
                    swait(b)

                pltpu.async_copy(ones_k, acc.at[src_v.at[j]], sems[b],
                                 add=True)
            return carry

        lax.fori_loop(0, NCHUNK // 4, outer, 0)
        for b in range(4):
            swait(b)
        pltpu.sync_copy(ones_k, acc.at[src_v.at[NCHUNK - 1]], add=True)
        plsc.subcore_barrier()
        _writeout(zrows, acc, [ones_v, pbuf], row0, cid, out0, out1,
                  [sems[0], sems[1]])

    return deg


_BN = 1000
_PREC = jax.lax.Precision.DEFAULT


def _node_spec(d):
    return pl.BlockSpec((_BN, d), lambda i: (i, 0))


def _finish_body(dp0, dp1, v, dinv16, xsa, xsb):
    deg = dp0[:, 0:1] + dp1[:, 0:1]
    di = jnp.where(deg > 0, lax.rsqrt(jnp.maximum(deg, 1e-12)), 0.0)
    dinv16[...] = jnp.broadcast_to(di, (_BN, 16))
    d = v.shape[1] // 2
    xsa[...] = di * v[:, :d]
    xsb[...] = di * v[:, d:]


def _deg_finish(dp0, dp1, v):
    d = v.shape[1]
    return pl.pallas_call(
        _finish_body,
        grid=(N // _BN,),
        in_specs=[_node_spec(DH), _node_spec(DH), _node_spec(d)],
        out_specs=[_node_spec(16), _node_spec(d // 2), _node_spec(d // 2)],
        out_shape=[
            jax.ShapeDtypeStruct((N, 16), jnp.float32),
            jax.ShapeDtypeStruct((N, d // 2), jnp.float32),
            jax.ShapeDtypeStruct((N, d // 2), jnp.float32),
        ],
    )(dp0, dp1, v)


def _mid_body(combine, t1a, t1b, dinv16, *outs):
    di = dinv16[:, 0:1]
    s = -(di * di)
    if combine == "concat":
        outs[0][...] = s * t1a[...]
        outs[1][...] = s * t1b[...]
    else:
        outs[0][...] = s * (t1a[...] + t1b[...])


def _mid(t1a, t1b, dinv16, combine):
    dh = t1a.shape[1]
    n_out = 2 if combine == "concat" else 1
    return pl.pallas_call(
        functools.partial(_mid_body, combine),
        grid=(N // _BN,),
        in_specs=[_node_spec(dh), _node_spec(dh), _node_spec(16)],
        out_specs=[_node_spec(dh)] * n_out,
        out_shape=[jax.ShapeDtypeStruct((N, dh), jnp.float32)] * n_out,
    )(t1a, t1b, dinv16)


def _layer_body(combine, emit, x, t1a, t1b, t2a, t2b, dinv16,
                w0, w1, w2, b, *outs):
    di = dinv16[:, 0:1]
    if combine == "concat":
        t1 = jnp.concatenate([t1a[...], t1b[...]], axis=1)
        t2 = jnp.concatenate([t2a[...], t2b[...]], axis=1)
    else:
        t1 = t1a[...] + t1b[...]
        t2 = t2a[...] + t2b[...]
    out = jnp.dot(x[...], w0[...] - w2[...], precision=_PREC)
    out += jnp.dot(t1 * (-di), w1[...], precision=_PREC)
    out += jnp.dot(t2 * (-2.0 * di), w2[...], precision=_PREC)
    out = jax.nn.relu(out + b[...])
    outs[0][...] = out
    if emit == "halves":
        d = out.shape[1] // 2
        xs = di * out
        outs[1][...] = xs[:, :d]
        outs[2][...] = xs[:, d:]
    elif emit == "full":
        outs[1][...] = di * out


def _layer(x, t1a, t1b, t2a, t2b, dinv16, W, b, combine, emit):
    din = x.shape[1]
    dh = t1a.shape[1]
    dout = W.shape[2]
    b2 = b.reshape(1, dout)
    wspec = pl.BlockSpec((din, dout), lambda i: (0, 0))
    out_specs = [_node_spec(dout)]
    out_shape = [jax.ShapeDtypeStruct((N, dout), jnp.float32)]
    if emit == "halves":
        out_specs += [_node_spec(dout // 2)] * 2
        out_shape += [jax.ShapeDtypeStruct((N, dout // 2), jnp.float32)] * 2
    elif emit == "full":
        out_specs += [_node_spec(dout)]
        out_shape += [jax.ShapeDtypeStruct((N, dout), jnp.float32)]
    return pl.pallas_call(
        functools.partial(_layer_body, combine, emit),
        grid=(N // _BN,),
        in_specs=[
            _node_spec(din),
            _node_spec(dh), _node_spec(dh), _node_spec(dh), _node_spec(dh),
            _node_spec(16),
            wspec, wspec, wspec,
            pl.BlockSpec((1, dout), lambda i: (0, 0)),
        ],
        out_specs=out_specs,
        out_shape=out_shape,
    )(x, t1a, t1b, t2a, t2b, dinv16, W[0], W[1], W[2], b2)


def kernel(v, edge_index, W1, b1, W2, b2, W3, b3):
    src = edge_index[0]
    dst = edge_index[1]

    ones_rows = jnp.ones((KD, DH), jnp.float32)
    zrows = jnp.zeros((K, DH), jnp.float32)

    prop_col = _make_prop_col()
    prop_edge = _make_prop_edge()

    npad = EP - E
    src_p = jnp.concatenate([src, jnp.zeros((npad,), jnp.int32)])
    dst_p = jnp.concatenate(
        [dst, N + (jnp.arange(npad, dtype=jnp.int32) % (NP - N))])
    sd16 = jnp.stack([src_p.reshape(NS, EP // (NS * K), K),
                      dst_p.reshape(NS, EP // (NS * K), K)], axis=2)
    sd32 = jnp.stack([src_p.reshape(NC * NS, EP // (NC * NS * K), K),
                      dst_p.reshape(NC * NS, EP // (NC * NS * K), K)], axis=2)

    degsrc = src.reshape(NC * NS, E // (NC * NS * KD), KD)
    dp0, dp1 = _make_deg()(degsrc, ones_rows, zrows)
    dinv16, xa, xb = _deg_finish(dp0, dp1, v)

    t1a, t1b = prop_col(xa, xb, sd16, zrows)
    ysa, ysb = _mid(t1a, t1b, dinv16, "concat")
    t2a, t2b = prop_col(ysa, ysb, sd16, zrows)
    x1, xs1 = _layer(v, t1a, t1b, t2a, t2b, dinv16, W1, b1, "concat", "full")

    p0, p1 = prop_edge(xs1, sd32, zrows)
    (ys,) = _mid(p0, p1, dinv16, "sum")
    q0, q1 = prop_edge(ys, sd32, zrows)
    x2, xa2, xb2 = _layer(x1, p0, p1, q0, q1, dinv16, W2, b2, "sum", "halves")

    u1a, u1b = prop_col(xa2, xb2, sd16, zrows)
    wsa, wsb = _mid(u1a, u1b, dinv16, "concat")
    u2a, u2b = prop_col(wsa, wsb, sd16, zrows)
    (x3,) = _layer(x2, u1a, u1b, u2a, u2b, dinv16, W3, b3, "concat", None)
    return x3


# speedup vs baseline: 1.4015x; 1.4015x over previous
import functools

import jax
import jax.numpy as jnp
from jax import lax
from jax.experimental import pallas as pl
from jax.experimental.pallas import tpu as pltpu
from jax.experimental.pallas import tpu_sc as plsc

N = 10000
NP = 10240
E = 160000
NC = 2
NS = 16
DH = 128

RPT = NP // NS
K = 40
EP = 160000
KD = 40
RB = 5

_MESH = dict(core_axis_name="c", subcore_axis_name="s")


def _prop_pipeline(nchunk, zrows, dummy_idx, acc, src_idx, dst_idx, rows,
                   isems, gsems, ssems, idx_load, gather):
    def gwait(b):
        pltpu.make_async_copy(zrows, rows[b], gsems[b]).wait()

    def iwait(b):
        pltpu.make_async_copy(dummy_idx, src_idx[b], isems[b]).wait()
        pltpu.make_async_copy(dummy_idx, dst_idx[b], isems[b]).wait()

    def swait(b):
        pltpu.make_async_copy(zrows, rows[b], ssems[b]).wait()

    for t in range(3):
        idx_load(t, t)
    for t in range(2):
        iwait(t)
        gather(t, t)

    def outer(ii, carry):
        for b in range(RB):
            j = ii * RB + b
            gwait(b)
            pltpu.async_copy(rows[b], acc.at[dst_idx[b]], ssems[b], add=True)

            @pl.when(j >= 2)
            def _():
                swait((b - 2) % RB)

            @pl.when(j + 3 < nchunk)
            def _():
                idx_load(j + 3, (b + 3) % RB)

            @pl.when(j + 2 < nchunk)
            def _():
                iwait((b + 2) % RB)
                gather(j + 2, (b + 2) % RB)

            return_val = carry
        return return_val

    lax.fori_loop(0, nchunk // RB, outer, 0)
    swait((nchunk - 2) % RB)
    swait((nchunk - 1) % RB)


def _zero_acc(zrows, rows0, acc, row0, sem):
    pltpu.sync_copy(zrows, rows0)
    nz = RPT // K
    for jj in range(nz):
        pltpu.async_copy(rows0, acc.at[pl.ds(row0 + jj * K, K)], sem)
    for jj in range(nz):
        pltpu.make_async_copy(zrows, rows0, sem).wait()


def _writeout(zrows, acc, rows, row0, cid, out0, out1, wsems):
    nw = RPT // K
    for jj in range(nw):
        b = jj & 1
        sl = pl.ds(row0 + jj * K, K)
        if jj >= 2:
            pltpu.make_async_copy(zrows, rows[b], wsems[b]).wait()
        pltpu.sync_copy(acc.at[sl], rows[b])

        @pl.when(cid == 0)
        def _():
            pltpu.async_copy(rows[b], out0.at[sl], wsems[b])

        @pl.when(cid == 1)
        def _():
            pltpu.async_copy(rows[b], out1.at[sl], wsems[b])

    for b in range(2):
        pltpu.make_async_copy(zrows, rows[b], wsems[b]).wait()


def _sc_scratch(nbuf=RB):
    return [
        pltpu.VMEM_SHARED((NP, DH), jnp.float32),
        [pltpu.VMEM((K,), jnp.int32)] * nbuf,
        [pltpu.VMEM((K,), jnp.int32)] * nbuf,
        [pltpu.VMEM((K, DH), jnp.float32)] * nbuf,
        [pltpu.SemaphoreType.DMA] * nbuf,
        [pltpu.SemaphoreType.DMA] * nbuf,
        [pltpu.SemaphoreType.DMA] * nbuf,
    ]


_OUT2 = (
    jax.ShapeDtypeStruct((NP, DH), jnp.float32),
    jax.ShapeDtypeStruct((NP, DH), jnp.float32),
)


@functools.lru_cache(maxsize=None)
def _make_prop_col():
    EPT = EP // NS
    NCHUNK = EPT // K

    @functools.partial(
        pl.kernel, out_type=_OUT2, mesh=plsc.VectorSubcoreMesh(**_MESH),
        scratch_types=_sc_scratch(),
    )
    def prop(xa, xb, src1, dst1, zrows, outa, outb,
             acc, src_idx, dst_idx, rows, isems, gsems, ssems):
        tid = lax.axis_index("s")
        cid = lax.axis_index("c")
        row0 = tid * RPT
        base_e = tid * EPT

        _zero_acc(zrows, rows[0], acc, row0, ssems[0])
        plsc.subcore_barrier()

        def idx_load(j, b):
            off = base_e + j * K
            pltpu.async_copy(src1.at[pl.ds(off, K)], src_idx[b], isems[b])
            pltpu.async_copy(dst1.at[pl.ds(off, K)], dst_idx[b], isems[b])

        def gather(j, b):
            @pl.when(cid == 0)
            def _():
                pltpu.async_copy(xa.at[src_idx[b]], rows[b], gsems[b])

            @pl.when(cid == 1)
            def _():
                pltpu.async_copy(xb.at[src_idx[b]], rows[b], gsems[b])

        _prop_pipeline(NCHUNK, zrows, src1.at[pl.ds(0, K)], acc, src_idx,
                       dst_idx, rows, isems, gsems, ssems, idx_load, gather)
        plsc.subcore_barrier()
        _writeout(zrows, acc, rows, row0, cid, outa, outb, gsems)

    return prop


@functools.lru_cache(maxsize=None)
def _make_prop_edge():
    EPT = EP // (NC * NS)
    NCHUNK = EPT // K

    @functools.partial(
        pl.kernel, out_type=_OUT2, mesh=plsc.VectorSubcoreMesh(**_MESH),
        scratch_types=_sc_scratch(),
    )
    def prop(x, src1, dst1, zrows, out0, out1,
             acc, src_idx, dst_idx, rows, isems, gsems, ssems):
        tid = lax.axis_index("s")
        cid = lax.axis_index("c")
        row0 = tid * RPT
        wid = cid * NS + tid
        base_e = wid * EPT

        _zero_acc(zrows, rows[0], acc, row0, ssems[0])
        plsc.subcore_barrier()

        def idx_load(j, b):
            off = base_e + j * K
            pltpu.async_copy(src1.at[pl.ds(off, K)], src_idx[b], isems[b])
            pltpu.async_copy(dst1.at[pl.ds(off, K)], dst_idx[b], isems[b])

        def gather(j, b):
            pltpu.async_copy(x.at[src_idx[b]], rows[b], gsems[b])

        _prop_pipeline(NCHUNK, zrows, src1.at[pl.ds(0, K)], acc, src_idx,
                       dst_idx, rows, isems, gsems, ssems, idx_load, gather)
        plsc.subcore_barrier()
        _writeout(zrows, acc, rows, row0, cid, out0, out1, gsems)

    return prop


@functools.lru_cache(maxsize=None)
def _make_deg():
    EPT = E // (NC * NS)
    NCHUNK = EPT // KD

    @functools.partial(
        pl.kernel, out_type=_OUT2, mesh=plsc.VectorSubcoreMesh(**_MESH),
        scratch_types=[
            pltpu.VMEM_SHARED((NP, DH), jnp.float32),
            pltpu.VMEM((NCHUNK, KD), jnp.int32),
            pltpu.VMEM((K, DH), jnp.float32),
            pltpu.VMEM((K, DH), jnp.float32),
            pltpu.VMEM((KD, DH), jnp.float32),
            [pltpu.SemaphoreType.DMA] * 4,
        ],
    )
    def deg(src3, ones_hbm, zrows, out0, out1,
            acc, src_v, ones_v, pbuf, ones_k, sems):
        tid = lax.axis_index("s")
        cid = lax.axis_index("c")
        row0 = tid * RPT
        wid = cid * NS + tid

        _zero_acc(zrows, ones_v, acc, row0, sems[0])
        pltpu.sync_copy(src3.at[wid], src_v)
        pltpu.sync_copy(ones_hbm, ones_k)
        plsc.subcore_barrier()

        def swait(b):
            pltpu.make_async_copy(ones_hbm, ones_k, sems[b]).wait()

        def outer(ii, carry):
            for b in range(4):
                j = ii * 4 + b

                @pl.when(j >= 4)
                def _():
                    swait(b)

                pltpu.async_copy(ones_k, acc.at[src_v.at[j]], sems[b],
                                 add=True)
            return carry

        lax.fori_loop(0, NCHUNK // 4, outer, 0)
        for b in range(4):
            swait(b)
        pltpu.sync_copy(ones_k, acc.at[src_v.at[NCHUNK - 1]], add=True)
        plsc.subcore_barrier()
        _writeout(zrows, acc, [ones_v, pbuf], row0, cid, out0, out1,
                  [sems[0], sems[1]])

    return deg


_BN = 1000
_PREC = jax.lax.Precision.DEFAULT


def _node_spec(d):
    return pl.BlockSpec((_BN, d), lambda i: (i, 0))


def _finish_body(dp0, dp1, v, dinv16, xsa, xsb):
    deg = dp0[:, 0:1] + dp1[:, 0:1]
    di = jnp.where(deg > 0, lax.rsqrt(jnp.maximum(deg, 1e-12)), 0.0)
    dinv16[...] = jnp.broadcast_to(di, (_BN, 16))
    d = v.shape[1] // 2
    xsa[...] = di * v[:, :d]
    xsb[...] = di * v[:, d:]


def _deg_finish(dp0, dp1, v):
    d = v.shape[1]
    return pl.pallas_call(
        _finish_body,
        grid=(N // _BN,),
        in_specs=[_node_spec(DH), _node_spec(DH), _node_spec(d)],
        out_specs=[_node_spec(16), _node_spec(d // 2), _node_spec(d // 2)],
        out_shape=[
            jax.ShapeDtypeStruct((N, 16), jnp.float32),
            jax.ShapeDtypeStruct((N, d // 2), jnp.float32),
            jax.ShapeDtypeStruct((N, d // 2), jnp.float32),
        ],
    )(dp0, dp1, v)


def _mid_body(combine, t1a, t1b, dinv16, *outs):
    di = dinv16[:, 0:1]
    s = -(di * di)
    if combine == "concat":
        outs[0][...] = s * t1a[...]
        outs[1][...] = s * t1b[...]
    else:
        outs[0][...] = s * (t1a[...] + t1b[...])


def _mid(t1a, t1b, dinv16, combine):
    dh = t1a.shape[1]
    n_out = 2 if combine == "concat" else 1
    return pl.pallas_call(
        functools.partial(_mid_body, combine),
        grid=(N // _BN,),
        in_specs=[_node_spec(dh), _node_spec(dh), _node_spec(16)],
        out_specs=[_node_spec(dh)] * n_out,
        out_shape=[jax.ShapeDtypeStruct((N, dh), jnp.float32)] * n_out,
    )(t1a, t1b, dinv16)


def _layer_body(combine, emit, x, t1a, t1b, t2a, t2b, dinv16,
                w0, w1, w2, b, *outs):
    di = dinv16[:, 0:1]
    if combine == "concat":
        t1 = jnp.concatenate([t1a[...], t1b[...]], axis=1)
        t2 = jnp.concatenate([t2a[...], t2b[...]], axis=1)
    else:
        t1 = t1a[...] + t1b[...]
        t2 = t2a[...] + t2b[...]
    out = jnp.dot(x[...], w0[...] - w2[...], precision=_PREC)
    out += jnp.dot(t1 * (-di), w1[...], precision=_PREC)
    out += jnp.dot(t2 * (-2.0 * di), w2[...], precision=_PREC)
    out = jax.nn.relu(out + b[...])
    outs[0][...] = out
    if emit == "halves":
        d = out.shape[1] // 2
        xs = di * out
        outs[1][...] = xs[:, :d]
        outs[2][...] = xs[:, d:]
    elif emit == "full":
        outs[1][...] = di * out


def _layer(x, t1a, t1b, t2a, t2b, dinv16, W, b, combine, emit):
    din = x.shape[1]
    dh = t1a.shape[1]
    dout = W.shape[2]
    b2 = b.reshape(1, dout)
    wspec = pl.BlockSpec((din, dout), lambda i: (0, 0))
    out_specs = [_node_spec(dout)]
    out_shape = [jax.ShapeDtypeStruct((N, dout), jnp.float32)]
    if emit == "halves":
        out_specs += [_node_spec(dout // 2)] * 2
        out_shape += [jax.ShapeDtypeStruct((N, dout // 2), jnp.float32)] * 2
    elif emit == "full":
        out_specs += [_node_spec(dout)]
        out_shape += [jax.ShapeDtypeStruct((N, dout), jnp.float32)]
    return pl.pallas_call(
        functools.partial(_layer_body, combine, emit),
        grid=(N // _BN,),
        in_specs=[
            _node_spec(din),
            _node_spec(dh), _node_spec(dh), _node_spec(dh), _node_spec(dh),
            _node_spec(16),
            wspec, wspec, wspec,
            pl.BlockSpec((1, dout), lambda i: (0, 0)),
        ],
        out_specs=out_specs,
        out_shape=out_shape,
    )(x, t1a, t1b, t2a, t2b, dinv16, W[0], W[1], W[2], b2)


def kernel(v, edge_index, W1, b1, W2, b2, W3, b3):
    src = edge_index[0]
    dst = edge_index[1]
    ones_rows = jnp.ones((KD, DH), jnp.float32)
    zrows = jnp.zeros((K, DH), jnp.float32)

    prop_col = _make_prop_col()
    prop_edge = _make_prop_edge()

    degsrc = src.reshape(NC * NS, E // (NC * NS * KD), KD)
    dp0, dp1 = _make_deg()(degsrc, ones_rows, zrows)
    dinv16, xa, xb = _deg_finish(dp0, dp1, v)

    t1a, t1b = prop_col(xa, xb, src, dst, zrows)
    ysa, ysb = _mid(t1a, t1b, dinv16, "concat")
    t2a, t2b = prop_col(ysa, ysb, src, dst, zrows)
    x1, xs1 = _layer(v, t1a, t1b, t2a, t2b, dinv16, W1, b1, "concat", "full")

    p0, p1 = prop_edge(xs1, src, dst, zrows)
    (ys,) = _mid(p0, p1, dinv16, "sum")
    q0, q1 = prop_edge(ys, src, dst, zrows)
    x2, xa2, xb2 = _layer(x1, p0, p1, q0, q1, dinv16, W2, b2, "sum", "halves")

    u1a, u1b = prop_col(xa2, xb2, src, dst, zrows)
    wsa, wsb = _mid(u1a, u1b, dinv16, "concat")
    u2a, u2b = prop_col(wsa, wsb, src, dst, zrows)
    (x3,) = _layer(x2, u1a, u1b, u2a, u2b, dinv16, W3, b3, "concat", None)
    return x3
